# baseline (speedup 1.0000x reference)
import functools

import jax
import jax.numpy as jnp
from jax import lax
from jax.experimental import pallas as pl
from jax.experimental.pallas import tpu as pltpu
from jax.experimental.pallas import tpu_sc as plsc

_USER = 60000
_ITEM = 40000
_N = _USER + _ITEM
_E = 1200000
_D = 64

_NC = 2
_NS = 16
_NR = 4
_R = _N // _NR
_DUMP = _R
_ACC_ROWS = _R + 8
_ZROWS = 1568
_WROWS = 1664

_B = 128
_NQ = 2
_CH = 2048
_CBUF = _CH + _B
_CROWS = _CBUF // _B
_NCHUNK = -(-_E // _CH)
_EPAD = _NCHUNK * _CH
_KMAX = -(-_NCHUNK // _NS)


def _vrsqrt(x):
    xi = lax.bitcast_convert_type(x, jnp.int32)
    yi = jnp.int32(0x5F3759DF) - lax.shift_right_logical(xi, 1)
    y = lax.bitcast_convert_type(yi, jnp.float32)
    for _ in range(3):
        y = y * (1.5 - 0.5 * x * y * y)
    return y


def _lane_bcast(v, lane):
    idx = jnp.full((16, 1), lane, jnp.int32)
    return lax.gather(
        v, idx,
        lax.GatherDimensionNumbers(
            offset_dims=(), collapsed_slice_dims=(0,), start_index_map=(0,)),
        slice_sizes=(1,),
        mode=lax.GatherScatterMode.PROMISE_IN_BOUNDS)


def _spmm_body(ego, src, dst, w, zhbm, out,
               srcb, dstb, wb, csrc, cdst, cw, rows, acc,
               gsem0, gsem1, ssem):
    gsems = (gsem0, gsem1)
    cid = lax.axis_index("c")
    sid = lax.axis_index("s")

    zi = jnp.zeros((16,), jnp.int32)

    def zinit_body(i, c):
        for cvec in range(_B // 16):
            csrc[i, pl.ds(cvec * 16, 16)] = zi
            cdst[i, pl.ds(cvec * 16, 16)] = zi
        return c

    lax.fori_loop(0, _CROWS, zinit_body, 0)

    for ri in range(2):
        rng = 2 * cid + ri
        lo = rng * _R

        pltpu.sync_copy(zhbm, rows.at[0])
        zb = jnp.minimum(sid * _ZROWS, _ACC_ROWS - _ZROWS)
        for k in range(_ZROWS // _B):
            pltpu.sync_copy(rows.at[0], acc.at[pl.ds(zb + k * _B, _B)])
        rem = _ZROWS % _B
        pltpu.sync_copy(rows.at[0].at[pl.ds(0, rem)],
                        acc.at[pl.ds(zb + (_ZROWS // _B) * _B, rem)])
        plsc.subcore_barrier()

        def chunk_body(k, carry):
            g = sid + k * _NS

            @pl.when(g < _NCHUNK)
            def _():
                base = g * _CH
                pltpu.sync_copy(src.at[pl.ds(base, _CH)], srcb)
                pltpu.sync_copy(dst.at[pl.ds(base, _CH)], dstb)
                pltpu.sync_copy(w.at[pl.ds(base, _CH)], wb)

                def scan_body(i, cntv):
                    sv = srcb[pl.ds(i * 16, 16)]
                    dv = dstb[pl.ds(i * 16, 16)]
                    wv = wb[pl.ds(i * 16, 16)]
                    m = (dv >= lo) & (dv < lo + _R)
                    mi = jnp.where(m, 1, 0).astype(jnp.int32)
                    incl = plsc.cumsum(mi)
                    pos = cntv + incl - mi
                    ph = lax.shift_right_logical(pos, 7)
                    plo = pos & (_B - 1)
                    plsc.store_scatter(csrc, [ph, plo], sv, mask=m)
                    plsc.store_scatter(cdst, [ph, plo], dv - lo, mask=m)
                    plsc.store_scatter(cw, [ph, plo], wv, mask=m)
                    return cntv + plsc.all_reduce_population_count(m)

                cntv = lax.fori_loop(0, _CH // 16, scan_body,
                                     jnp.zeros((16,), jnp.int32),
                                     unroll=4)
                cnt = jnp.max(cntv)

                zw = jnp.zeros((16,), jnp.float32)
                iot = lax.iota(jnp.int32, 16)
                for p in range(_B // 16):
                    posv = cnt + p * 16 + iot
                    plsc.store_scatter(
                        cw, [lax.shift_right_logical(posv, 7),
                             posv & (_B - 1)], zw)

                nb = lax.shift_right_logical(cnt + (_B - 1), 7)
                nrb = lax.div(nb + (_NQ - 1), jnp.int32(_NQ))

                def round_body(rb, c2):
                    for q in range(_NQ):
                        bi = rb * _NQ + q

                        @pl.when(bi < nb)
                        def _(bi=bi, q=q):
                            pltpu.async_copy(
                                ego.at[csrc.at[bi]], rows.at[q], gsems[q])

                    for q in range(_NQ):
                        bi = rb * _NQ + q

                        @pl.when(bi < nb)
                        def _(bi=bi, q=q):
                            pltpu.make_async_copy(
                                ego.at[csrc.at[bi]], rows.at[q],
                                gsems[q]).wait()

                            for i in range(_B // 16):
                                wv = cw[bi, pl.ds(i * 16, 16)]
                                for lane in range(16):
                                    ws = _lane_bcast(wv, lane)
                                    r = i * 16 + lane
                                    for j in range(_D // 16):
                                        rows[q, r, pl.ds(j * 16, 16)] = (
                                            rows[q, r, pl.ds(j * 16, 16)]
                                            * ws)

                            pltpu.async_copy(rows.at[q], acc.at[cdst.at[bi]],
                                             ssem, add=True)

                    for q in range(_NQ):
                        bi = rb * _NQ + q

                        @pl.when(bi < nb)
                        def _(bi=bi, q=q):
                            pltpu.make_async_copy(
                                rows.at[q], acc.at[cdst.at[bi]], ssem).wait()

                    return c2

                lax.fori_loop(0, nrb, round_body, 0)

            return carry

        lax.fori_loop(0, _KMAX, chunk_body, 0)
        plsc.subcore_barrier()

        wbase = jnp.minimum(sid * _WROWS, _R - _WROWS)

        def wb_body(kb, c4):
            pltpu.sync_copy(acc.at[pl.ds(wbase + kb * _B, _B)], rows.at[0])

            def nrm_body(r, c5):
                v0 = rows[0, r, pl.ds(0, 16)]
                v1 = rows[0, r, pl.ds(16, 16)]
                v2 = rows[0, r, pl.ds(32, 16)]
                v3 = rows[0, r, pl.ds(48, 16)]
                p = v0 * v0 + v1 * v1 + v2 * v2 + v3 * v3
                ss = _lane_bcast(plsc.cumsum(p), 15)
                y = _vrsqrt(ss)
                rows[0, r, pl.ds(0, 16)] = v0 * y
                rows[0, r, pl.ds(16, 16)] = v1 * y
                rows[0, r, pl.ds(32, 16)] = v2 * y
                rows[0, r, pl.ds(48, 16)] = v3 * y
                return c5

            lax.fori_loop(0, _B, nrm_body, 0)
            pltpu.sync_copy(rows.at[0],
                            out.at[pl.ds(lo + wbase + kb * _B, _B)])
            return c4

        lax.fori_loop(0, _WROWS // _B, wb_body, 0)
        plsc.subcore_barrier()


_spmm = functools.partial(
    pl.kernel,
    mesh=plsc.VectorSubcoreMesh(core_axis_name="c", subcore_axis_name="s"),
    out_type=jax.ShapeDtypeStruct((_N, _D), jnp.float32),
    compiler_params=pltpu.CompilerParams(
        use_tc_tiling_on_sc=False, needs_layout_passes=False),
    scratch_types=[
        pltpu.VMEM((_CH,), jnp.int32),
        pltpu.VMEM((_CH,), jnp.int32),
        pltpu.VMEM((_CH,), jnp.float32),
        pltpu.VMEM((_CROWS, _B), jnp.int32),
        pltpu.VMEM((_CROWS, _B), jnp.int32),
        pltpu.VMEM((_CROWS, _B), jnp.float32),
        pltpu.VMEM((_NQ, _B, _D), jnp.float32),
        pltpu.VMEM_SHARED((_ACC_ROWS, _D), jnp.float32),
        pltpu.SemaphoreType.DMA,
        pltpu.SemaphoreType.DMA,
        pltpu.SemaphoreType.DMA,
    ],
)(_spmm_body)


def _mean_body(a_ref, b_ref, c_ref, d_ref, o_ref):
    o_ref[...] = 0.25 * (a_ref[...] + b_ref[...] + c_ref[...] + d_ref[...])


_mean = pl.pallas_call(
    _mean_body,
    grid=(100,),
    in_specs=[pl.BlockSpec((_N // 100, _D), lambda i: (i, 0))] * 4,
    out_specs=pl.BlockSpec((_N // 100, _D), lambda i: (i, 0)),
    out_shape=jax.ShapeDtypeStruct((_N, _D), jnp.float32),
)


def kernel(user_emb, item_emb, edge_index, edge_weight):
    ego0 = jnp.concatenate([user_emb, item_emb], axis=0)
    pad = _EPAD - _E
    src = jnp.concatenate([edge_index[0], jnp.zeros((pad,), jnp.int32)])
    dst = jnp.concatenate(
        [edge_index[1], jnp.full((pad,), jnp.int32(2**30))])
    w = jnp.concatenate([edge_weight, jnp.zeros((pad,), jnp.float32)])
    zeros = jnp.zeros((_B, _D), jnp.float32)

    layers = [ego0]
    e = ego0
    for _ in range(3):
        e = _spmm(e, src, dst, w, zeros)
        layers.append(e)

    avg = _mean(*layers)
    return avg[:_USER], avg[_USER:]


# speedup vs baseline: 1.0013x; 1.0013x over previous
import functools

import jax
import jax.numpy as jnp
from jax import lax
from jax.experimental import pallas as pl
from jax.experimental.pallas import tpu as pltpu
from jax.experimental.pallas import tpu_sc as plsc

_USER = 60000
_ITEM = 40000
_N = _USER + _ITEM
_E = 1200000
_D = 64

_NC = 2
_NS = 16
_NR = 4
_R = _N // _NR
_DUMP = _R
_ACC_ROWS = _R + 8
_ZROWS = 1568
_WROWS = 1664

_B = 128
_NQ = 2
_CH = 2048
_CBUF = _CH + _B
_CROWS = _CBUF // _B
_NCHUNK = -(-_E // _CH)
_EPAD = _NCHUNK * _CH
_KMAX = -(-_NCHUNK // _NS)


def _vrsqrt(x):
    xi = lax.bitcast_convert_type(x, jnp.int32)
    yi = jnp.int32(0x5F3759DF) - lax.shift_right_logical(xi, 1)
    y = lax.bitcast_convert_type(yi, jnp.float32)
    for _ in range(3):
        y = y * (1.5 - 0.5 * x * y * y)
    return y


def _lane_bcast(v, lane):
    idx = jnp.full((16, 1), lane, jnp.int32)
    return lax.gather(
        v, idx,
        lax.GatherDimensionNumbers(
            offset_dims=(), collapsed_slice_dims=(0,), start_index_map=(0,)),
        slice_sizes=(1,),
        mode=lax.GatherScatterMode.PROMISE_IN_BOUNDS)


def _spmm_body(ego, src, dst, w, zhbm, out,
               srcb, dstb, wb, csrc, cdst, cw, rows, acc,
               gsem0, gsem1, ssem):
    gsems = (gsem0, gsem1)
    cid = lax.axis_index("c")
    sid = lax.axis_index("s")

    zi = jnp.zeros((16,), jnp.int32)

    def zinit_body(i, c):
        for cvec in range(_B // 16):
            csrc[i, pl.ds(cvec * 16, 16)] = zi
            cdst[i, pl.ds(cvec * 16, 16)] = zi
        return c

    lax.fori_loop(0, _CROWS, zinit_body, 0)

    for ri in range(2):
        rng = 2 * cid + ri
        lo = rng * _R

        pltpu.sync_copy(zhbm, rows.at[0])
        zb = jnp.minimum(sid * _ZROWS, _ACC_ROWS - _ZROWS)
        for k in range(_ZROWS // _B):
            pltpu.sync_copy(rows.at[0], acc.at[pl.ds(zb + k * _B, _B)])
        rem = _ZROWS % _B
        pltpu.sync_copy(rows.at[0].at[pl.ds(0, rem)],
                        acc.at[pl.ds(zb + (_ZROWS // _B) * _B, rem)])
        plsc.subcore_barrier()

        def chunk_body(k, carry):
            g = sid + k * _NS

            @pl.when(g < _NCHUNK)
            def _():
                base = g * _CH
                pltpu.sync_copy(src.at[pl.ds(base, _CH)], srcb)
                pltpu.sync_copy(dst.at[pl.ds(base, _CH)], dstb)
                pltpu.sync_copy(w.at[pl.ds(base, _CH)], wb)

                def scan_body(i, cntv):
                    sv = srcb[pl.ds(i * 16, 16)]
                    dv = dstb[pl.ds(i * 16, 16)]
                    wv = wb[pl.ds(i * 16, 16)]
                    m = (dv >= lo) & (dv < lo + _R)
                    mi = jnp.where(m, 1, 0).astype(jnp.int32)
                    incl = plsc.cumsum(mi)
                    pos = cntv + incl - mi
                    ph = lax.shift_right_logical(pos, 7)
                    plo = pos & (_B - 1)
                    plsc.store_scatter(csrc, [ph, plo], sv, mask=m)
                    plsc.store_scatter(cdst, [ph, plo], dv - lo, mask=m)
                    plsc.store_scatter(cw, [ph, plo], wv, mask=m)
                    return cntv + plsc.all_reduce_population_count(m)

                cntv = lax.fori_loop(0, _CH // 16, scan_body,
                                     jnp.zeros((16,), jnp.int32))
                cnt = jnp.max(cntv)

                zw = jnp.zeros((16,), jnp.float32)
                iot = lax.iota(jnp.int32, 16)
                for p in range(_B // 16):
                    posv = cnt + p * 16 + iot
                    plsc.store_scatter(
                        cw, [lax.shift_right_logical(posv, 7),
                             posv & (_B - 1)], zw)

                nb = lax.shift_right_logical(cnt + (_B - 1), 7)
                nrb = lax.div(nb + (_NQ - 1), jnp.int32(_NQ))

                def round_body(rb, c2):
                    for q in range(_NQ):
                        bi = rb * _NQ + q

                        @pl.when(bi < nb)
                        def _(bi=bi, q=q):
                            pltpu.async_copy(
                                ego.at[csrc.at[bi]], rows.at[q], gsems[q])

                    for q in range(_NQ):
                        bi = rb * _NQ + q

                        @pl.when(bi < nb)
                        def _(bi=bi, q=q):
                            pltpu.make_async_copy(
                                ego.at[csrc.at[bi]], rows.at[q],
                                gsems[q]).wait()

                            for i in range(_B // 16):
                                wv = cw[bi, pl.ds(i * 16, 16)]
                                for lane in range(16):
                                    ws = _lane_bcast(wv, lane)
                                    r = i * 16 + lane
                                    for j in range(_D // 16):
                                        rows[q, r, pl.ds(j * 16, 16)] = (
                                            rows[q, r, pl.ds(j * 16, 16)]
                                            * ws)

                            pltpu.async_copy(rows.at[q], acc.at[cdst.at[bi]],
                                             ssem, add=True)

                    for q in range(_NQ):
                        bi = rb * _NQ + q

                        @pl.when(bi < nb)
                        def _(bi=bi, q=q):
                            pltpu.make_async_copy(
                                rows.at[q], acc.at[cdst.at[bi]], ssem).wait()

                    return c2

                lax.fori_loop(0, nrb, round_body, 0)

            return carry

        lax.fori_loop(0, _KMAX, chunk_body, 0)
        plsc.subcore_barrier()

        wbase = jnp.minimum(sid * _WROWS, _R - _WROWS)

        def wb_body(kb, c4):
            pltpu.sync_copy(acc.at[pl.ds(wbase + kb * _B, _B)], rows.at[0])

            def nrm_body(r, c5):
                v0 = rows[0, r, pl.ds(0, 16)]
                v1 = rows[0, r, pl.ds(16, 16)]
                v2 = rows[0, r, pl.ds(32, 16)]
                v3 = rows[0, r, pl.ds(48, 16)]
                p = v0 * v0 + v1 * v1 + v2 * v2 + v3 * v3
                ss = _lane_bcast(plsc.cumsum(p), 15)
                y = _vrsqrt(ss)
                rows[0, r, pl.ds(0, 16)] = v0 * y
                rows[0, r, pl.ds(16, 16)] = v1 * y
                rows[0, r, pl.ds(32, 16)] = v2 * y
                rows[0, r, pl.ds(48, 16)] = v3 * y
                return c5

            lax.fori_loop(0, _B, nrm_body, 0)
            pltpu.sync_copy(rows.at[0],
                            out.at[pl.ds(lo + wbase + kb * _B, _B)])
            return c4

        lax.fori_loop(0, _WROWS // _B, wb_body, 0)
        plsc.subcore_barrier()


_spmm = functools.partial(
    pl.kernel,
    mesh=plsc.VectorSubcoreMesh(core_axis_name="c", subcore_axis_name="s"),
    out_type=jax.ShapeDtypeStruct((_N, _D), jnp.float32),
    compiler_params=pltpu.CompilerParams(
        use_tc_tiling_on_sc=False, needs_layout_passes=False),
    scratch_types=[
        pltpu.VMEM((_CH,), jnp.int32),
        pltpu.VMEM((_CH,), jnp.int32),
        pltpu.VMEM((_CH,), jnp.float32),
        pltpu.VMEM((_CROWS, _B), jnp.int32),
        pltpu.VMEM((_CROWS, _B), jnp.int32),
        pltpu.VMEM((_CROWS, _B), jnp.float32),
        pltpu.VMEM((_NQ, _B, _D), jnp.float32),
        pltpu.VMEM_SHARED((_ACC_ROWS, _D), jnp.float32),
        pltpu.SemaphoreType.DMA,
        pltpu.SemaphoreType.DMA,
        pltpu.SemaphoreType.DMA,
    ],
)(_spmm_body)


def _mean_body(a_ref, b_ref, c_ref, d_ref, o_ref):
    o_ref[...] = 0.25 * (a_ref[...] + b_ref[...] + c_ref[...] + d_ref[...])


_mean = pl.pallas_call(
    _mean_body,
    grid=(100,),
    in_specs=[pl.BlockSpec((_N // 100, _D), lambda i: (i, 0))] * 4,
    out_specs=pl.BlockSpec((_N // 100, _D), lambda i: (i, 0)),
    out_shape=jax.ShapeDtypeStruct((_N, _D), jnp.float32),
)


def kernel(user_emb, item_emb, edge_index, edge_weight):
    ego0 = jnp.concatenate([user_emb, item_emb], axis=0)
    pad = _EPAD - _E
    src = jnp.concatenate([edge_index[0], jnp.zeros((pad,), jnp.int32)])
    dst = jnp.concatenate(
        [edge_index[1], jnp.full((pad,), jnp.int32(2**30))])
    w = jnp.concatenate([edge_weight, jnp.zeros((pad,), jnp.float32)])
    zeros = jnp.zeros((_B, _D), jnp.float32)

    layers = [ego0]
    e = ego0
    for _ in range(3):
        e = _spmm(e, src, dst, w, zeros)
        layers.append(e)

    avg = _mean(*layers)
    return avg[:_USER], avg[_USER:]


# speedup vs baseline: 2.3167x; 2.3138x over previous
import functools

import jax
import jax.numpy as jnp
from jax import lax
from jax.experimental import pallas as pl
from jax.experimental.pallas import tpu as pltpu
from jax.experimental.pallas import tpu_sc as plsc

_USER = 60000
_ITEM = 40000
_N = _USER + _ITEM
_E = 1200000
_D = 64

_NC = 2
_NS = 16
_NR = 4
_R = _N // _NR
_DUMP = _R
_ACC_ROWS = _R + 8
_ZROWS = 1568
_WROWS = 1664

_B = 128
_NQ = 2
_CH = 1920
_CBUF = _CH + _B
_CROWS = _CBUF // _B
_NCHUNK = -(-_E // _CH)
_EPAD = _NCHUNK * _CH
_KMAX = -(-_NCHUNK // _NS)


def _vrsqrt(x):
    xi = lax.bitcast_convert_type(x, jnp.int32)
    yi = jnp.int32(0x5F3759DF) - lax.shift_right_logical(xi, 1)
    y = lax.bitcast_convert_type(yi, jnp.float32)
    for _ in range(3):
        y = y * (1.5 - 0.5 * x * y * y)
    return y


def _lane_bcast(v, lane):
    idx = jnp.full((16, 1), lane, jnp.int32)
    return lax.gather(
        v, idx,
        lax.GatherDimensionNumbers(
            offset_dims=(), collapsed_slice_dims=(0,), start_index_map=(0,)),
        slice_sizes=(1,),
        mode=lax.GatherScatterMode.PROMISE_IN_BOUNDS)


def _spmm_body(ego, src, dst, w, zhbm, out,
               srcb, dstb, wb, csrc, cdst, cw, rows, acc,
               gsem0, gsem1, ssem):
    gsems = (gsem0, gsem1)
    cid = lax.axis_index("c")
    sid = lax.axis_index("s")

    zi = jnp.zeros((16,), jnp.int32)

    def zinit_body(i, c):
        for cvec in range(_B // 16):
            csrc[i, pl.ds(cvec * 16, 16)] = zi
            cdst[i, pl.ds(cvec * 16, 16)] = zi
        return c

    lax.fori_loop(0, _CROWS, zinit_body, 0)

    for ri in range(2):
        rng = 2 * cid + ri
        lo = rng * _R

        pltpu.sync_copy(zhbm, rows.at[0])
        zb = jnp.minimum(sid * _ZROWS, _ACC_ROWS - _ZROWS)
        for k in range(_ZROWS // _B):
            pltpu.sync_copy(rows.at[0], acc.at[pl.ds(zb + k * _B, _B)])
        rem = _ZROWS % _B
        pltpu.sync_copy(rows.at[0].at[pl.ds(0, rem)],
                        acc.at[pl.ds(zb + (_ZROWS // _B) * _B, rem)])
        plsc.subcore_barrier()

        def chunk_body(k, carry):
            g = sid + k * _NS

            @pl.when(g < _NCHUNK)
            def _():
                base = g * _CH
                pltpu.sync_copy(src.at[pl.ds(base, _CH)], srcb)
                pltpu.sync_copy(dst.at[pl.ds(base, _CH)], dstb)
                pltpu.sync_copy(w.at[pl.ds(base, _CH)], wb)

                def scan_body(i, cntv):
                    sv = srcb[pl.ds(i * 16, 16)]
                    dv = dstb[pl.ds(i * 16, 16)]
                    wv = wb[pl.ds(i * 16, 16)]
                    m = (dv >= lo) & (dv < lo + _R)
                    mi = jnp.where(m, 1, 0).astype(jnp.int32)
                    incl = plsc.cumsum(mi)
                    pos = cntv + incl - mi
                    ph = lax.shift_right_logical(pos, 7)
                    plo = pos & (_B - 1)
                    plsc.store_scatter(csrc, [ph, plo], sv, mask=m)
                    plsc.store_scatter(cdst, [ph, plo], dv - lo, mask=m)
                    plsc.store_scatter(cw, [ph, plo], wv, mask=m)
                    return cntv + plsc.all_reduce_population_count(m)

                cntv = lax.fori_loop(0, _CH // 16, scan_body,
                                     jnp.zeros((16,), jnp.int32))
                cnt = jnp.max(cntv)

                zw = jnp.zeros((16,), jnp.float32)
                iot = lax.iota(jnp.int32, 16)
                for p in range(_B // 16):
                    posv = cnt + p * 16 + iot
                    plsc.store_scatter(
                        cw, [lax.shift_right_logical(posv, 7),
                             posv & (_B - 1)], zw)

                nb = lax.shift_right_logical(cnt + (_B - 1), 7)
                nrb = lax.div(nb + (_NQ - 1), jnp.int32(_NQ))

                def round_body(rb, c2):
                    for q in range(_NQ):
                        bi = rb * _NQ + q

                        @pl.when(bi < nb)
                        def _(bi=bi, q=q):
                            pltpu.async_copy(
                                ego.at[csrc.at[bi]], rows.at[q], gsems[q])

                    for q in range(_NQ):
                        bi = rb * _NQ + q

                        @pl.when(bi < nb)
                        def _(bi=bi, q=q):
                            pltpu.make_async_copy(
                                ego.at[csrc.at[bi]], rows.at[q],
                                gsems[q]).wait()

                            for i in range(_B // 16):
                                wv = cw[bi, pl.ds(i * 16, 16)]
                                for lane in range(16):
                                    ws = _lane_bcast(wv, lane)
                                    r = i * 16 + lane
                                    for j in range(_D // 16):
                                        rows[q, r, pl.ds(j * 16, 16)] = (
                                            rows[q, r, pl.ds(j * 16, 16)]
                                            * ws)

                            pltpu.async_copy(rows.at[q], acc.at[cdst.at[bi]],
                                             ssem, add=True)

                    for q in range(_NQ):
                        bi = rb * _NQ + q

                        @pl.when(bi < nb)
                        def _(bi=bi, q=q):
                            pltpu.make_async_copy(
                                rows.at[q], acc.at[cdst.at[bi]], ssem).wait()

                    return c2

                lax.fori_loop(0, nrb, round_body, 0)

            return carry

        lax.fori_loop(0, _KMAX, chunk_body, 0)
        plsc.subcore_barrier()

        wbase = jnp.minimum(sid * _WROWS, _R - _WROWS)

        def wb_body(kb, c4):
            pltpu.sync_copy(acc.at[pl.ds(wbase + kb * _B, _B)], rows.at[0])

            def nrm_body(r, c5):
                v0 = rows[0, r, pl.ds(0, 16)]
                v1 = rows[0, r, pl.ds(16, 16)]
                v2 = rows[0, r, pl.ds(32, 16)]
                v3 = rows[0, r, pl.ds(48, 16)]
                p = v0 * v0 + v1 * v1 + v2 * v2 + v3 * v3
                ss = _lane_bcast(plsc.cumsum(p), 15)
                y = _vrsqrt(ss)
                rows[0, r, pl.ds(0, 16)] = v0 * y
                rows[0, r, pl.ds(16, 16)] = v1 * y
                rows[0, r, pl.ds(32, 16)] = v2 * y
                rows[0, r, pl.ds(48, 16)] = v3 * y
                return c5

            lax.fori_loop(0, _B, nrm_body, 0)
            pltpu.sync_copy(rows.at[0],
                            out.at[pl.ds(lo + wbase + kb * _B, _B)])
            return c4

        lax.fori_loop(0, _WROWS // _B, wb_body, 0)
        plsc.subcore_barrier()


_spmm = functools.partial(
    pl.kernel,
    mesh=plsc.VectorSubcoreMesh(core_axis_name="c", subcore_axis_name="s"),
    out_type=jax.ShapeDtypeStruct((_N, _D), jnp.float32),
    compiler_params=pltpu.CompilerParams(
        use_tc_tiling_on_sc=False, needs_layout_passes=False),
    scratch_types=[
        pltpu.VMEM((_CH,), jnp.int32),
        pltpu.VMEM((_CH,), jnp.int32),
        pltpu.VMEM((_CH,), jnp.float32),
        pltpu.VMEM((_CROWS, _B), jnp.int32),
        pltpu.VMEM((_CROWS, _B), jnp.int32),
        pltpu.VMEM((_CROWS, _B), jnp.float32),
        pltpu.VMEM((_NQ, _B, _D), jnp.float32),
        pltpu.VMEM_SHARED((_ACC_ROWS, _D), jnp.float32),
        pltpu.SemaphoreType.DMA,
        pltpu.SemaphoreType.DMA,
        pltpu.SemaphoreType.DMA,
    ],
)(_spmm_body)


def _mean_body(a_ref, b_ref, c_ref, d_ref, o_ref):
    o_ref[...] = 0.25 * (a_ref[...] + b_ref[...] + c_ref[...] + d_ref[...])


_mean = pl.pallas_call(
    _mean_body,
    grid=(100,),
    in_specs=[pl.BlockSpec((_N // 100, _D), lambda i: (i, 0))] * 4,
    out_specs=pl.BlockSpec((_N // 100, _D), lambda i: (i, 0)),
    out_shape=jax.ShapeDtypeStruct((_N, _D), jnp.float32),
)


def kernel(user_emb, item_emb, edge_index, edge_weight):
    ego0 = jnp.concatenate([user_emb, item_emb], axis=0)
    pad = _EPAD - _E
    if pad:
        src = jnp.concatenate([edge_index[0],
                               jnp.zeros((pad,), jnp.int32)])
        dst = jnp.concatenate(
            [edge_index[1], jnp.full((pad,), jnp.int32(2**30))])
        w = jnp.concatenate([edge_weight,
                             jnp.zeros((pad,), jnp.float32)])
    else:
        src, dst, w = edge_index[0], edge_index[1], edge_weight
    zeros = jnp.zeros((_B, _D), jnp.float32)

    layers = [ego0]
    e = ego0
    for _ in range(3):
        e = _spmm(e, src, dst, w, zeros)
        layers.append(e)

    avg = _mean(*layers)
    return avg[:_USER], avg[_USER:]
